# Initial kernel scaffold; baseline (speedup 1.0000x reference)
import jax
import jax.numpy as jnp
from jax.experimental import pallas as pl


def kernel(x, w_qkv, w_dw, w_proj, temperature, a1, a2, a3, a4):
    raise NotImplementedError("write your pallas kernel here")


# speedup vs baseline: 4.1519x; 4.1519x over previous
import functools

import jax
import jax.numpy as jnp
from jax.experimental import pallas as pl
from jax.experimental.pallas import tpu as pltpu

C_IN = 96
H = 384
W = 384
N = H * W
TH = 16
L_OUT = TH * W
L_EXT = (TH + 2) * W
NTILES = H // TH
_BIG = 1e30
_HIGH = jax.lax.Precision.HIGHEST


def _dwconv_flat(y_ext, dw_ref, nchan):
    zcol = jnp.zeros((nchan, 1), jnp.float32)
    ypad = jnp.concatenate([zcol, y_ext, zcol], axis=1)
    wpos = jax.lax.broadcasted_iota(jnp.int32, (1, L_OUT), 1) % W
    left_ok = (wpos != 0).astype(jnp.float32)
    right_ok = (wpos != W - 1).astype(jnp.float32)
    acc = jnp.zeros((nchan, L_OUT), jnp.float32)
    for dx in range(3):
        part = jnp.zeros((nchan, L_OUT), jnp.float32)
        for dy in range(3):
            o = dy * W + dx
            tap = dw_ref[:, 3 * dy + dx : 3 * dy + dx + 1]
            part = part + tap * ypad[:, o : o + L_OUT]
        if dx == 0:
            part = part * left_ok
        elif dx == 2:
            part = part * right_ok
        acc = acc + part
    return acc


def _pass_a_body(x_mid, x_top, x_bot, wqk_ref, dw_ref, s_ref, nrm_ref):
    i = pl.program_id(0)
    tfac = jnp.where(i == 0, 0.0, 1.0).astype(jnp.float32)
    bfac = jnp.where(i == NTILES - 1, 0.0, 1.0).astype(jnp.float32)
    ext = jnp.concatenate(
        [x_top[...] * tfac, x_mid[...], x_bot[...] * bfac], axis=1)
    y = jax.lax.dot_general(
        wqk_ref[...], ext, (((1,), (0,)), ((), ())),
        preferred_element_type=jnp.float32)
    qk = _dwconv_flat(y, dw_ref, 2 * C_IN)
    q = qk[:C_IN, :]
    k = qk[C_IN:, :]
    s_part = jax.lax.dot_general(
        q, k, (((1,), (1,)), ((), ())),
        preferred_element_type=jnp.float32)
    rsq = jnp.sum(q * q, axis=1).reshape(1, C_IN)
    rsk = jnp.sum(k * k, axis=1).reshape(1, C_IN)
    nrm_part = jnp.concatenate([rsq, rsk], axis=0)

    @pl.when(i == 0)
    def _():
        s_ref[...] = s_part
        nrm_ref[...] = nrm_part

    @pl.when(i != 0)
    def _():
        s_ref[...] += s_part
        nrm_ref[...] += nrm_part


def _pass_b_body(s_ref, nrm_ref, temp_ref, aa_ref, p_ref, m_ref):
    s = s_ref[...]
    inv_q = jax.lax.rsqrt(jnp.maximum(nrm_ref[0:1, :], 1e-24))
    inv_k = jax.lax.rsqrt(jnp.maximum(nrm_ref[1:2, :], 1e-24))
    rows = jax.lax.broadcasted_iota(jnp.int32, (C_IN, C_IN), 0)
    cols = jax.lax.broadcasted_iota(jnp.int32, (C_IN, C_IN), 1)
    hper = C_IN // 2
    t0 = temp_ref[0, 0]
    t1 = temp_ref[0, 1]
    trow = jnp.where(rows < hper, t0, t1)
    attn = s * inv_q.reshape(C_IN, 1) * inv_k.reshape(1, C_IN) * trow
    headmask = (rows // hper) == (cols // hper)
    am = jnp.where(headmask, attn, -_BIG)
    ai = am[:, :, None]
    aj = am[:, None, :]
    jlt = (jax.lax.broadcasted_iota(jnp.int32, (1, C_IN, C_IN), 2)
           < jax.lax.broadcasted_iota(jnp.int32, (1, C_IN, C_IN), 1))
    beats = (aj > ai) | ((aj == ai) & jlt)
    rank = jnp.sum(beats.astype(jnp.float32), axis=2)
    a_comb = jnp.zeros((C_IN, C_IN), jnp.float32)
    for bi, kk in enumerate((hper // 2, hper * 2 // 3, hper * 3 // 4,
                             hper * 4 // 5)):
        m = (rank < kk) & headmask
        amk = jnp.where(m, attn, -_BIG)
        rmax = jnp.max(amk, axis=1, keepdims=True)
        e = jnp.exp(amk - rmax) * m.astype(jnp.float32)
        sm = e / jnp.sum(e, axis=1, keepdims=True)
        a_comb = a_comb + aa_ref[0, bi] * sm
    m_ref[...] = jax.lax.dot_general(
        p_ref[...], a_comb, (((1,), (0,)), ((), ())),
        preferred_element_type=jnp.float32)


def _pass_c_body(x_mid, x_top, x_bot, wv_ref, dw_ref, m_ref, out_ref):
    i = pl.program_id(0)
    tfac = jnp.where(i == 0, 0.0, 1.0).astype(jnp.float32)
    bfac = jnp.where(i == NTILES - 1, 0.0, 1.0).astype(jnp.float32)
    ext = jnp.concatenate(
        [x_top[...] * tfac, x_mid[...], x_bot[...] * bfac], axis=1)
    y = jax.lax.dot_general(
        wv_ref[...], ext, (((1,), (0,)), ((), ())),
        preferred_element_type=jnp.float32)
    v = _dwconv_flat(y, dw_ref, C_IN)
    out_ref[...] = jax.lax.dot_general(
        m_ref[...], v, (((1,), (0,)), ((), ())),
        preferred_element_type=jnp.float32)


@functools.partial(jax.jit, static_argnames=())
def kernel(x, w_qkv, w_dw, w_proj, temperature, a1, a2, a3, a4):
    xf = x.reshape(C_IN, N)
    wqk = w_qkv[: 2 * C_IN, :, 0, 0]
    wv = w_qkv[2 * C_IN :, :, 0, 0]
    dwqk = w_dw[: 2 * C_IN, 0].reshape(2 * C_IN, 9)
    dwv = w_dw[2 * C_IN :, 0].reshape(C_IN, 9)
    p = w_proj[:, :, 0, 0]
    temp = temperature.reshape(1, 2)
    aa = jnp.concatenate([a1, a2, a3, a4]).reshape(1, 4)

    x_mid_spec = pl.BlockSpec((C_IN, L_OUT), lambda i: (0, i))
    x_top_spec = pl.BlockSpec(
        (C_IN, W), lambda i: (0, jnp.maximum(i * TH - 1, 0)))
    x_bot_spec = pl.BlockSpec(
        (C_IN, W), lambda i: (0, jnp.minimum((i + 1) * TH, H - 1)))
    full = lambda shape: pl.BlockSpec(shape, lambda i: (0, 0))

    s, nrm = pl.pallas_call(
        _pass_a_body,
        grid=(NTILES,),
        in_specs=[x_mid_spec, x_top_spec, x_bot_spec,
                  full((2 * C_IN, C_IN)), full((2 * C_IN, 9))],
        out_specs=[full((C_IN, C_IN)), full((2, C_IN))],
        out_shape=[jax.ShapeDtypeStruct((C_IN, C_IN), jnp.float32),
                   jax.ShapeDtypeStruct((2, C_IN), jnp.float32)],
        compiler_params=pltpu.CompilerParams(
            dimension_semantics=("arbitrary",)),
    )(xf, xf, xf, wqk, dwqk)

    m = pl.pallas_call(
        _pass_b_body,
        out_shape=jax.ShapeDtypeStruct((C_IN, C_IN), jnp.float32),
    )(s, nrm, temp, aa, p)

    out = pl.pallas_call(
        _pass_c_body,
        grid=(NTILES,),
        in_specs=[x_mid_spec, x_top_spec, x_bot_spec,
                  full((C_IN, C_IN)), full((C_IN, 9)), full((C_IN, C_IN))],
        out_specs=pl.BlockSpec((C_IN, L_OUT), lambda i: (0, i)),
        out_shape=jax.ShapeDtypeStruct((C_IN, N), jnp.float32),
        compiler_params=pltpu.CompilerParams(
            dimension_semantics=("arbitrary",)),
    )(xf, xf, xf, wv, dwv, m)

    return out.reshape(1, C_IN, H, W)
